# baseline (speedup 1.0000x reference)
import functools

import jax
import jax.numpy as jnp
from jax import lax
from jax.experimental import pallas as pl
from jax.experimental.pallas import tpu as pltpu
import jax.experimental.pallas.tpu_sc as plsc

E_NODES = 10000
D = 128
N_REL = 1000
GAMMA = 1.0

NW = 32
CHUNK = 64
SLAB = 16
NBUF = 4


def _pad2d(x, fill):
    e = x.shape[0]
    blk = NW * CHUNK * SLAB
    rows_needed = -(-e // blk) * blk
    if rows_needed != e:
        x = jnp.concatenate([x, jnp.full((rows_needed - e,), fill, x.dtype)])
    return x.reshape(rows_needed // CHUNK, CHUNK)


def _sc_mesh():
    return plsc.VectorSubcoreMesh(core_axis_name="c", subcore_axis_name="s")


def _scale_chunk(gbuf, b, valsl, j, dw=D):
    def scale(g, _):
        v16 = valsl[j, pl.ds(g * 16, 16)]
        for u in range(16):
            vv = jnp.full((16,), v16[u], jnp.float32)
            for q in range(dw // 16):
                sl = (b, g * 16 + u, pl.ds(q * 16, 16))
                gbuf[sl] = gbuf[sl] * vv
        return 0
    lax.fori_loop(0, CHUNK // 16, scale, 0)


def _edge_pipeline(w, nc, nbuf, rows_h, cols_h, vals_h, table_r, table_h, acc,
                   colsl, rowsl, valsl, gbuf, gsems, ssems, dw=D):
    la = nbuf // 2
    nslab = nc // SLAB

    def slab_body(sl, _):
        off = w * nc + sl * SLAB
        pltpu.sync_copy(cols_h.at[pl.ds(off, SLAB)], colsl)
        pltpu.sync_copy(vals_h.at[pl.ds(off, SLAB)], valsl)
        pltpu.sync_copy(rows_h.at[pl.ds(off, SLAB)], rowsl)
        for p in range(la):
            pltpu.async_copy(table_r.at[colsl.at[p]], gbuf.at[p], gsems[p])

        def group(g, _):
            for b in range(nbuf):
                j = g * nbuf + b
                bn = (b + la) % nbuf

                @pl.when(j >= la)
                def _():
                    pltpu.make_async_copy(
                        gbuf.at[bn], acc.at[pl.ds(0, CHUNK)], ssems[bn]).wait()

                @pl.when(j < SLAB - la)
                def _():
                    pltpu.async_copy(
                        table_r.at[colsl.at[j + la]], gbuf.at[bn], gsems[bn])

                pltpu.make_async_copy(
                    table_h.at[pl.ds(0, CHUNK)], gbuf.at[b], gsems[b]).wait()
                _scale_chunk(gbuf, b, valsl, j, dw)
                pltpu.async_copy(
                    gbuf.at[b], acc.at[rowsl.at[j]], ssems[b], add=True)
            return 0

        lax.fori_loop(0, SLAB // nbuf, group, 0)
        for jt in range(SLAB - la, SLAB):
            bt = jt % nbuf
            pltpu.make_async_copy(
                gbuf.at[bt], acc.at[pl.ds(0, CHUNK)], ssems[bt]).wait()
        return 0

    lax.fori_loop(0, nslab, slab_body, 0)


def _partitioned_copy(src, dst, s, n_rows):
    nsplit = max(k for k in range(1, 17) if n_rows % k == 0 and (n_rows // k) % 8 == 0)
    rp = n_rows // nsplit

    @pl.when(s < nsplit)
    def _():
        pltpu.sync_copy(src.at[pl.ds(s * rp, rp)], dst.at[pl.ds(s * rp, rp)])


def _spmm_scratch(nbuf):
    return [
        pltpu.VMEM((SLAB, CHUNK), jnp.int32),
        pltpu.VMEM((SLAB, CHUNK), jnp.int32),
        pltpu.VMEM((SLAB, CHUNK), jnp.float32),
        pltpu.VMEM((nbuf, CHUNK, D), jnp.float32),
    ] + [pltpu.SemaphoreType.DMA] * (2 * nbuf)


def _spmm_sc(rows2d, cols2d, vals2d, table, n_out, zeros, stage_table):
    nr = rows2d.shape[0]
    nc = nr // NW
    n_src = table.shape[0]
    nbuf = 2 if stage_table else 4
    scratch = [pltpu.VMEM_SHARED((n_out, D), jnp.float32)]
    if stage_table:
        scratch.append(pltpu.VMEM_SHARED((n_src, D), jnp.float32))
    scratch += _spmm_scratch(nbuf)

    @functools.partial(
        pl.kernel,
        out_type=jax.ShapeDtypeStruct((2, n_out, D), jnp.float32),
        mesh=_sc_mesh(),
        scratch_types=scratch,
    )
    def k(rows_h, cols_h, vals_h, table_h, zeros_h, out_h, acc, *rest):
        if stage_table:
            table_sp = rest[0]
            rest = rest[1:]
        colsl, rowsl, valsl, gbuf = rest[:4]
        sems = rest[4:]
        gsems, ssems = sems[:nbuf], sems[nbuf:]
        c = lax.axis_index("c")
        s = lax.axis_index("s")
        w = c * 16 + s
        _partitioned_copy(zeros_h, acc, s, n_out)
        if stage_table:
            _partitioned_copy(table_h, table_sp, s, n_src)
            table_r = table_sp
        else:
            table_r = table_h
        plsc.subcore_barrier()
        _edge_pipeline(w, nc, nbuf, rows_h, cols_h, vals_h, table_r, table_h, acc,
                       colsl, rowsl, valsl, gbuf, gsems, ssems)
        plsc.subcore_barrier()

        @pl.when(s == 0)
        def _():
            pltpu.sync_copy(acc, out_h.at[c])

    return k(rows2d, cols2d, vals2d, table, zeros)


def _headtail_sc(hr, hc, hv, tr, tc, tv, we, zeros_rel):
    nr = hr.shape[0]
    nc = nr // NW
    nbuf = 2

    @functools.partial(
        pl.kernel,
        out_type=jax.ShapeDtypeStruct((2, 2, N_REL, D), jnp.float32),
        mesh=_sc_mesh(),
        scratch_types=[
            pltpu.VMEM_SHARED((N_REL, D), jnp.float32),
            pltpu.VMEM_SHARED((N_REL, D), jnp.float32),
            pltpu.VMEM_SHARED((E_NODES, D), jnp.float32),
        ] + _spmm_scratch(nbuf),
    )
    def k(hr_h, hc_h, hv_h, tr_h, tc_h, tv_h, we_h, z_h, out_h,
          accl, accr, we_sp, colsl, rowsl, valsl, gbuf, *sems):
        gsems, ssems = sems[:nbuf], sems[nbuf:]
        c = lax.axis_index("c")
        s = lax.axis_index("s")
        w = c * 16 + s

        @pl.when(s == 14)
        def _():
            pltpu.sync_copy(z_h, accl)

        @pl.when(s == 15)
        def _():
            pltpu.sync_copy(z_h, accr)
        _partitioned_copy(we_h, we_sp, s, E_NODES)
        plsc.subcore_barrier()

        _edge_pipeline(w, nc, nbuf, hr_h, hc_h, hv_h, we_sp, we_h, accl,
                       colsl, rowsl, valsl, gbuf, gsems, ssems)
        _edge_pipeline(w, nc, nbuf, tr_h, tc_h, tv_h, we_sp, we_h, accr,
                       colsl, rowsl, valsl, gbuf, gsems, ssems)
        plsc.subcore_barrier()

        @pl.when(s == 0)
        def _():
            pltpu.sync_copy(accl, out_h.at[c, 0])

        @pl.when(s == 1)
        def _():
            pltpu.sync_copy(accr, out_h.at[c, 1])

    return k(hr, hc, hv, tr, tc, tv, we, zeros_rel)


def _gather_sc(idx2d, node):
    nr = idx2d.shape[0]
    nc = nr // NW
    nbuf = 4
    la = nbuf // 2

    @functools.partial(
        pl.kernel,
        out_type=jax.ShapeDtypeStruct((nr * CHUNK, D), jnp.float32),
        mesh=_sc_mesh(),
        scratch_types=[
            pltpu.VMEM_SHARED((E_NODES, D), jnp.float32),
            pltpu.VMEM((SLAB, CHUNK), jnp.int32),
            pltpu.VMEM((nbuf, CHUNK, D), jnp.float32),
        ] + [pltpu.SemaphoreType.DMA] * (2 * nbuf),
    )
    def k(idx_h, node_h, out_h, node_sp, colsl, gbuf, *sems):
        gsems, wsems = sems[:nbuf], sems[nbuf:]
        c = lax.axis_index("c")
        s = lax.axis_index("s")
        w = c * 16 + s
        nslab = nc // SLAB
        _partitioned_copy(node_h, node_sp, s, E_NODES)
        plsc.subcore_barrier()

        def slab_body(sl, _):
            off = w * nc + sl * SLAB
            pltpu.sync_copy(idx_h.at[pl.ds(off, SLAB)], colsl)
            for p in range(la):
                pltpu.async_copy(node_sp.at[colsl.at[p]], gbuf.at[p], gsems[p])

            def group(g, _):
                for b in range(nbuf):
                    j = g * nbuf + b
                    bn = (b + la) % nbuf

                    @pl.when(j >= la)
                    def _():
                        pltpu.make_async_copy(
                            gbuf.at[bn], out_h.at[pl.ds(0, CHUNK)], wsems[bn]).wait()

                    @pl.when(j < SLAB - la)
                    def _():
                        pltpu.async_copy(
                            node_sp.at[colsl.at[j + la]], gbuf.at[bn], gsems[bn])

                    pltpu.make_async_copy(
                        node_h.at[pl.ds(0, CHUNK)], gbuf.at[b], gsems[b]).wait()
                    pltpu.async_copy(
                        gbuf.at[b], out_h.at[pl.ds((off + j) * CHUNK, CHUNK)],
                        wsems[b])
                return 0

            lax.fori_loop(0, SLAB // nbuf, group, 0)
            for jt in range(SLAB - la, SLAB):
                bt = jt % nbuf
                pltpu.make_async_copy(
                    gbuf.at[bt], out_h.at[pl.ds(0, CHUNK)], wsems[bt]).wait()
            return 0

        lax.fori_loop(0, nslab, slab_body, 0)

    return k(idx2d, node)


def _tc_call(body, out_shapes, *args):
    return pl.pallas_call(
        body,
        out_shape=out_shapes,
    )(*args)


def _tc_norm_p(word_emb, d1, bias):
    def body(w_ref, d1_ref, b_ref, we_ref, p_ref):
        w = w_ref[...]
        norm = jnp.maximum(jnp.sqrt(jnp.sum(w * w, axis=-1, keepdims=True)), 1e-12)
        we = w / norm
        we_ref[...] = we
        p_ref[...] = jnp.dot(we, d1_ref[...], preferred_element_type=jnp.float32) + b_ref[...]

    return _tc_call(
        body,
        (jax.ShapeDtypeStruct((E_NODES, D), jnp.float32),
         jax.ShapeDtypeStruct((E_NODES, D), jnp.float32)),
        word_emb, d1, bias,
    )


def _tc_z(lr_part, d23a, d23b):
    def body(lr_ref, a_ref, b_ref, z_ref):
        l = lr_ref[0, 0] + lr_ref[1, 0]
        r = lr_ref[0, 1] + lr_ref[1, 1]
        z1 = (jnp.dot(l, a_ref[...], preferred_element_type=jnp.float32)
              + jnp.dot(r, b_ref[...], preferred_element_type=jnp.float32))
        z_ref[...] = jnp.concatenate([z1, -z1], axis=0)

    return _tc_call(
        body,
        jax.ShapeDtypeStruct((2 * N_REL, D), jnp.float32),
        lr_part, d23a, d23b,
    )


def _tc_nr(nb_part, we, p, w1, kg, bg):
    def body(nb_ref, we_ref, p_ref, w1_ref, kg_ref, bg_ref, nr_ref, x1_ref, t1_ref):
        neighbor = nb_ref[0] + nb_ref[1]
        nr = we_ref[...] + jax.nn.relu(p_ref[...] + neighbor)
        nr_ref[...] = nr
        x1_ref[...] = jnp.dot(nr, w1_ref[...], preferred_element_type=jnp.float32)
        t1_ref[...] = jax.nn.sigmoid(
            jnp.dot(nr, kg_ref[...], preferred_element_type=jnp.float32) + bg_ref[...])

    return _tc_call(
        body,
        (jax.ShapeDtypeStruct((E_NODES, D), jnp.float32),
         jax.ShapeDtypeStruct((E_NODES, D), jnp.float32),
         jax.ShapeDtypeStruct((E_NODES, D), jnp.float32)),
        nb_part, we, p, w1, kg, bg,
    )


def _tc_h1(s1_part, nr, t1, w2, kg, bg):
    def body(s1_ref, nr_ref, t1_ref, w2_ref, kg_ref, bg_ref, h1_ref, x2_ref, t2_ref):
        g1 = jax.nn.relu(s1_ref[0] + s1_ref[1])
        t1 = t1_ref[...]
        h1 = t1 * g1 + (1.0 - t1) * nr_ref[...]
        h1_ref[...] = h1
        x2_ref[...] = jnp.dot(h1, w2_ref[...], preferred_element_type=jnp.float32)
        t2_ref[...] = jax.nn.sigmoid(
            jnp.dot(h1, kg_ref[...], preferred_element_type=jnp.float32) + bg_ref[...])

    return _tc_call(
        body,
        (jax.ShapeDtypeStruct((E_NODES, D), jnp.float32),
         jax.ShapeDtypeStruct((E_NODES, D), jnp.float32),
         jax.ShapeDtypeStruct((E_NODES, D), jnp.float32)),
        s1_part, nr, t1, w2, kg, bg,
    )


def _tc_node(s2_part, h1, t2):
    def body(s2_ref, h1_ref, t2_ref, node_ref):
        g2 = jax.nn.relu(s2_ref[0] + s2_ref[1])
        t2 = t2_ref[...]
        node_ref[...] = t2 * g2 + (1.0 - t2) * h1_ref[...]

    return _tc_call(
        body,
        jax.ShapeDtypeStruct((E_NODES, D), jnp.float32),
        s2_part, h1, t2,
    )


L_OFF = 0
R_OFF = 5000
NR_OFF = 10000
NL_OFF = 85000


def _tc_loss(g, mask_col, T, K):
    TB = 200
    TBK = TB * K
    nblk = T // TB

    def body(l_ref, r_ref, nr_ref, nl_ref, m_ref, o_ref):
        i = pl.program_id(0)
        l = l_ref[...]
        r = r_ref[...]
        sel = (jax.lax.broadcasted_iota(jnp.int32, (TBK, TB), 0) // K
               == jax.lax.broadcasted_iota(jnp.int32, (TBK, TB), 1)).astype(jnp.float32)
        A = jnp.sum(jnp.abs(l - r), axis=1, keepdims=True)
        dm_rep = jnp.dot(sel, A + GAMMA, preferred_element_type=jnp.float32)
        l_rep = jnp.dot(sel, l, preferred_element_type=jnp.float32)
        r_rep = jnp.dot(sel, r, preferred_element_type=jnp.float32)
        B = jnp.sum(jnp.abs(l_rep - nr_ref[...]), axis=1, keepdims=True)
        B2 = jnp.sum(jnp.abs(nl_ref[...] - r_rep), axis=1, keepdims=True)
        m = m_ref[...]
        terms = (jax.nn.relu(dm_rep - B) + jax.nn.relu(dm_rep - B2)) * m
        part = (jnp.sum(terms) / 2.0).reshape(1, 1)

        @pl.when(i == 0)
        def _():
            o_ref[...] = jnp.zeros_like(o_ref)

        o_ref[...] += part

    out = pl.pallas_call(
        body,
        grid=(nblk,),
        in_specs=[
            pl.BlockSpec((TB, D), lambda i: (i + L_OFF // TB, 0)),
            pl.BlockSpec((TB, D), lambda i: (i + R_OFF // TB, 0)),
            pl.BlockSpec((TBK, D), lambda i: (i + NR_OFF // TBK, 0)),
            pl.BlockSpec((TBK, D), lambda i: (i + NL_OFF // TBK, 0)),
            pl.BlockSpec((TBK, 1), lambda i: (i, 0)),
        ],
        out_specs=pl.BlockSpec((1, 1), lambda i: (0, 0)),
        out_shape=jax.ShapeDtypeStruct((1, 1), jnp.float32),
    )(g, g, g, g, mask_col)
    return out[0, 0]


def kernel(left_idx, right_idx, neg_right, neg_left, head_rows, head_cols, head_vals, tail_rows, tail_cols, tail_vals, er_rows, er_cols, er_vals, adj_rows, adj_cols, adj_vals, mask, word_emb, kernel_gate, bias_gate, W1, W2, Dense, Bias):
    f32 = jnp.float32
    i32 = jnp.int32
    zeros_nodes = jnp.zeros((E_NODES, D), f32)
    zeros_rel = jnp.zeros((N_REL, D), f32)

    hr, hc, hv = _pad2d(head_rows.astype(i32), 0), _pad2d(head_cols.astype(i32), 0), _pad2d(head_vals, 0.0)
    tr, tc, tv = _pad2d(tail_rows.astype(i32), 0), _pad2d(tail_cols.astype(i32), 0), _pad2d(tail_vals, 0.0)
    err, erc, erv = _pad2d(er_rows.astype(i32), 0), _pad2d(er_cols.astype(i32), 0), _pad2d(er_vals, 0.0)
    ar, ac, av = _pad2d(adj_rows.astype(i32), 0), _pad2d(adj_cols.astype(i32), 0), _pad2d(adj_vals, 0.0)

    we, p = _tc_norm_p(word_emb, Dense[:D], Bias.reshape(1, D))

    lr_part = _headtail_sc(hr, hc, hv, tr, tc, tv, we, zeros_rel)
    z = _tc_z(lr_part, Dense[D:2 * D], Dense[2 * D:])

    nb_part = _spmm_sc(err, erc, erv, z, E_NODES, zeros_nodes, stage_table=True)
    nr, x1, t1 = _tc_nr(nb_part, we, p, W1, kernel_gate, bias_gate.reshape(1, D))

    s1_part = _spmm_sc(ar, ac, av, x1, E_NODES, zeros_nodes, stage_table=False)
    h1, x2, t2 = _tc_h1(s1_part, nr, t1, W2, kernel_gate, bias_gate.reshape(1, D))

    s2_part = _spmm_sc(ar, ac, av, x2, E_NODES, zeros_nodes, stage_table=False)
    node = _tc_node(s2_part, h1, t2)

    t_pairs, k_neg = neg_right.shape
    all_idx = jnp.zeros((NL_OFF + t_pairs * k_neg,), i32)
    all_idx = all_idx.at[L_OFF:L_OFF + t_pairs].set(left_idx.astype(i32))
    all_idx = all_idx.at[R_OFF:R_OFF + t_pairs].set(right_idx.astype(i32))
    all_idx = all_idx.at[NR_OFF:NR_OFF + t_pairs * k_neg].set(neg_right.astype(i32).reshape(-1))
    all_idx = all_idx.at[NL_OFF:].set(neg_left.astype(i32).reshape(-1))
    idx2d = _pad2d(all_idx, 0)
    g = _gather_sc(idx2d, node)
    return _tc_loss(g, mask.reshape(t_pairs * k_neg, 1), t_pairs, k_neg)


# speedup vs baseline: 1.0392x; 1.0392x over previous
import functools

import jax
import jax.numpy as jnp
from jax import lax
from jax.experimental import pallas as pl
from jax.experimental.pallas import tpu as pltpu
import jax.experimental.pallas.tpu_sc as plsc

E_NODES = 10000
D = 128
N_REL = 1000
GAMMA = 1.0

NW = 32
CHUNK = 64
SLAB = 16
NBUF = 4
SLAB_ST = 32


def _pad2d(x, fill, slab=SLAB):
    e = x.shape[0]
    blk = NW * CHUNK * slab
    rows_needed = -(-e // blk) * blk
    if rows_needed != e:
        x = jnp.concatenate([x, jnp.full((rows_needed - e,), fill, x.dtype)])
    return x.reshape(rows_needed // CHUNK, CHUNK)


def _sc_mesh():
    return plsc.VectorSubcoreMesh(core_axis_name="c", subcore_axis_name="s")


def _scale_chunk(gbuf, b, valsl, j, dw=D):
    def scale(g, _):
        v16 = valsl[j, pl.ds(g * 16, 16)]
        for u in range(16):
            vv = jnp.full((16,), v16[u], jnp.float32)
            for q in range(dw // 16):
                sl = (b, g * 16 + u, pl.ds(q * 16, 16))
                gbuf[sl] = gbuf[sl] * vv
        return 0
    lax.fori_loop(0, CHUNK // 16, scale, 0)


def _edge_pipeline(w, nc, nbuf, rows_h, cols_h, vals_h, table_r, table_h, acc,
                   colsl, rowsl, valsl, gbuf, gsems, ssems, dw=D, slab=SLAB):
    la = nbuf // 2
    nslab = nc // slab

    def slab_body(sl, _):
        off = w * nc + sl * slab
        pltpu.sync_copy(cols_h.at[pl.ds(off, slab)], colsl)
        pltpu.sync_copy(vals_h.at[pl.ds(off, slab)], valsl)
        pltpu.sync_copy(rows_h.at[pl.ds(off, slab)], rowsl)
        for p in range(la):
            pltpu.async_copy(table_r.at[colsl.at[p]], gbuf.at[p], gsems[p])

        def group(g, _):
            for b in range(nbuf):
                j = g * nbuf + b
                bn = (b + la) % nbuf

                @pl.when(j >= la)
                def _():
                    pltpu.make_async_copy(
                        gbuf.at[bn], acc.at[pl.ds(0, CHUNK)], ssems[bn]).wait()

                @pl.when(j < slab - la)
                def _():
                    pltpu.async_copy(
                        table_r.at[colsl.at[j + la]], gbuf.at[bn], gsems[bn])

                pltpu.make_async_copy(
                    table_h.at[pl.ds(0, CHUNK)], gbuf.at[b], gsems[b]).wait()
                _scale_chunk(gbuf, b, valsl, j, dw)
                pltpu.async_copy(
                    gbuf.at[b], acc.at[rowsl.at[j]], ssems[b], add=True)
            return 0

        lax.fori_loop(0, slab // nbuf, group, 0)
        for jt in range(slab - la, slab):
            bt = jt % nbuf
            pltpu.make_async_copy(
                gbuf.at[bt], acc.at[pl.ds(0, CHUNK)], ssems[bt]).wait()
        return 0

    lax.fori_loop(0, nslab, slab_body, 0)


def _partitioned_copy(src, dst, s, n_rows):
    nsplit = max(k for k in range(1, 17) if n_rows % k == 0 and (n_rows // k) % 8 == 0)
    rp = n_rows // nsplit

    @pl.when(s < nsplit)
    def _():
        pltpu.sync_copy(src.at[pl.ds(s * rp, rp)], dst.at[pl.ds(s * rp, rp)])


def _spmm_scratch(nbuf, slab=SLAB):
    return [
        pltpu.VMEM((slab, CHUNK), jnp.int32),
        pltpu.VMEM((slab, CHUNK), jnp.int32),
        pltpu.VMEM((slab, CHUNK), jnp.float32),
        pltpu.VMEM((nbuf, CHUNK, D), jnp.float32),
    ] + [pltpu.SemaphoreType.DMA] * (2 * nbuf)


def _spmm_sc(rows2d, cols2d, vals2d, table, n_out, zeros, stage_table):
    nr = rows2d.shape[0]
    nc = nr // NW
    n_src = table.shape[0]
    nbuf = 2 if stage_table else 4
    slab = SLAB_ST if stage_table else SLAB
    scratch = [pltpu.VMEM_SHARED((n_out, D), jnp.float32)]
    if stage_table:
        scratch.append(pltpu.VMEM_SHARED((n_src, D), jnp.float32))
    scratch += _spmm_scratch(nbuf, slab)

    @functools.partial(
        pl.kernel,
        out_type=jax.ShapeDtypeStruct((2, n_out, D), jnp.float32),
        mesh=_sc_mesh(),
        scratch_types=scratch,
    )
    def k(rows_h, cols_h, vals_h, table_h, zeros_h, out_h, acc, *rest):
        if stage_table:
            table_sp = rest[0]
            rest = rest[1:]
        colsl, rowsl, valsl, gbuf = rest[:4]
        sems = rest[4:]
        gsems, ssems = sems[:nbuf], sems[nbuf:]
        c = lax.axis_index("c")
        s = lax.axis_index("s")
        w = c * 16 + s
        _partitioned_copy(zeros_h, acc, s, n_out)
        if stage_table:
            _partitioned_copy(table_h, table_sp, s, n_src)
            table_r = table_sp
        else:
            table_r = table_h
        plsc.subcore_barrier()
        _edge_pipeline(w, nc, nbuf, rows_h, cols_h, vals_h, table_r, table_h, acc,
                       colsl, rowsl, valsl, gbuf, gsems, ssems, slab=slab)
        plsc.subcore_barrier()

        @pl.when(s == 0)
        def _():
            pltpu.sync_copy(acc, out_h.at[c])

    return k(rows2d, cols2d, vals2d, table, zeros)


def _headtail_sc(hr, hc, hv, tr, tc, tv, we, zeros_rel):
    nr = hr.shape[0]
    nc = nr // NW
    nbuf = 2
    slab = SLAB_ST

    @functools.partial(
        pl.kernel,
        out_type=jax.ShapeDtypeStruct((2, 2, N_REL, D), jnp.float32),
        mesh=_sc_mesh(),
        scratch_types=[
            pltpu.VMEM_SHARED((N_REL, D), jnp.float32),
            pltpu.VMEM_SHARED((N_REL, D), jnp.float32),
            pltpu.VMEM_SHARED((E_NODES, D), jnp.float32),
        ] + _spmm_scratch(nbuf, slab),
    )
    def k(hr_h, hc_h, hv_h, tr_h, tc_h, tv_h, we_h, z_h, out_h,
          accl, accr, we_sp, colsl, rowsl, valsl, gbuf, *sems):
        gsems, ssems = sems[:nbuf], sems[nbuf:]
        c = lax.axis_index("c")
        s = lax.axis_index("s")
        w = c * 16 + s

        @pl.when(s == 14)
        def _():
            pltpu.sync_copy(z_h, accl)

        @pl.when(s == 15)
        def _():
            pltpu.sync_copy(z_h, accr)
        _partitioned_copy(we_h, we_sp, s, E_NODES)
        plsc.subcore_barrier()

        _edge_pipeline(w, nc, nbuf, hr_h, hc_h, hv_h, we_sp, we_h, accl,
                       colsl, rowsl, valsl, gbuf, gsems, ssems, slab=slab)
        _edge_pipeline(w, nc, nbuf, tr_h, tc_h, tv_h, we_sp, we_h, accr,
                       colsl, rowsl, valsl, gbuf, gsems, ssems, slab=slab)
        plsc.subcore_barrier()

        @pl.when(s == 0)
        def _():
            pltpu.sync_copy(accl, out_h.at[c, 0])

        @pl.when(s == 1)
        def _():
            pltpu.sync_copy(accr, out_h.at[c, 1])

    return k(hr, hc, hv, tr, tc, tv, we, zeros_rel)


def _gather_sc(idx2d, node):
    nr = idx2d.shape[0]
    nc = nr // NW
    nbuf = 4
    la = nbuf // 2
    slab = SLAB

    @functools.partial(
        pl.kernel,
        out_type=jax.ShapeDtypeStruct((nr * CHUNK, D), jnp.float32),
        mesh=_sc_mesh(),
        scratch_types=[
            pltpu.VMEM_SHARED((E_NODES, D), jnp.float32),
            pltpu.VMEM((SLAB, CHUNK), jnp.int32),
            pltpu.VMEM((nbuf, CHUNK, D), jnp.float32),
        ] + [pltpu.SemaphoreType.DMA] * (2 * nbuf),
    )
    def k(idx_h, node_h, out_h, node_sp, colsl, gbuf, *sems):
        gsems, wsems = sems[:nbuf], sems[nbuf:]
        c = lax.axis_index("c")
        s = lax.axis_index("s")
        w = c * 16 + s
        nslab = nc // SLAB
        _partitioned_copy(node_h, node_sp, s, E_NODES)
        plsc.subcore_barrier()

        def slab_body(sl, _):
            off = w * nc + sl * SLAB
            pltpu.sync_copy(idx_h.at[pl.ds(off, SLAB)], colsl)
            for p in range(la):
                pltpu.async_copy(node_sp.at[colsl.at[p]], gbuf.at[p], gsems[p])

            def group(g, _):
                for b in range(nbuf):
                    j = g * nbuf + b
                    bn = (b + la) % nbuf

                    @pl.when(j >= la)
                    def _():
                        pltpu.make_async_copy(
                            gbuf.at[bn], out_h.at[pl.ds(0, CHUNK)], wsems[bn]).wait()

                    @pl.when(j < slab - la)
                    def _():
                        pltpu.async_copy(
                            node_sp.at[colsl.at[j + la]], gbuf.at[bn], gsems[bn])

                    pltpu.make_async_copy(
                        node_h.at[pl.ds(0, CHUNK)], gbuf.at[b], gsems[b]).wait()
                    pltpu.async_copy(
                        gbuf.at[b], out_h.at[pl.ds((off + j) * CHUNK, CHUNK)],
                        wsems[b])
                return 0

            lax.fori_loop(0, SLAB // nbuf, group, 0)
            for jt in range(SLAB - la, SLAB):
                bt = jt % nbuf
                pltpu.make_async_copy(
                    gbuf.at[bt], out_h.at[pl.ds(0, CHUNK)], wsems[bt]).wait()
            return 0

        lax.fori_loop(0, nslab, slab_body, 0)

    return k(idx2d, node)


def _tc_call(body, out_shapes, *args):
    return pl.pallas_call(
        body,
        out_shape=out_shapes,
    )(*args)


def _tc_norm_p(word_emb, d1, bias):
    def body(w_ref, d1_ref, b_ref, we_ref, p_ref):
        w = w_ref[...]
        norm = jnp.maximum(jnp.sqrt(jnp.sum(w * w, axis=-1, keepdims=True)), 1e-12)
        we = w / norm
        we_ref[...] = we
        p_ref[...] = jnp.dot(we, d1_ref[...], preferred_element_type=jnp.float32) + b_ref[...]

    return _tc_call(
        body,
        (jax.ShapeDtypeStruct((E_NODES, D), jnp.float32),
         jax.ShapeDtypeStruct((E_NODES, D), jnp.float32)),
        word_emb, d1, bias,
    )


def _tc_z(lr_part, d23a, d23b):
    def body(lr_ref, a_ref, b_ref, z_ref):
        l = lr_ref[0, 0] + lr_ref[1, 0]
        r = lr_ref[0, 1] + lr_ref[1, 1]
        z1 = (jnp.dot(l, a_ref[...], preferred_element_type=jnp.float32)
              + jnp.dot(r, b_ref[...], preferred_element_type=jnp.float32))
        z_ref[...] = jnp.concatenate([z1, -z1], axis=0)

    return _tc_call(
        body,
        jax.ShapeDtypeStruct((2 * N_REL, D), jnp.float32),
        lr_part, d23a, d23b,
    )


def _tc_nr(nb_part, we, p, w1, kg, bg):
    def body(nb_ref, we_ref, p_ref, w1_ref, kg_ref, bg_ref, nr_ref, x1_ref, t1_ref):
        neighbor = nb_ref[0] + nb_ref[1]
        nr = we_ref[...] + jax.nn.relu(p_ref[...] + neighbor)
        nr_ref[...] = nr
        x1_ref[...] = jnp.dot(nr, w1_ref[...], preferred_element_type=jnp.float32)
        t1_ref[...] = jax.nn.sigmoid(
            jnp.dot(nr, kg_ref[...], preferred_element_type=jnp.float32) + bg_ref[...])

    return _tc_call(
        body,
        (jax.ShapeDtypeStruct((E_NODES, D), jnp.float32),
         jax.ShapeDtypeStruct((E_NODES, D), jnp.float32),
         jax.ShapeDtypeStruct((E_NODES, D), jnp.float32)),
        nb_part, we, p, w1, kg, bg,
    )


def _tc_h1(s1_part, nr, t1, w2, kg, bg):
    def body(s1_ref, nr_ref, t1_ref, w2_ref, kg_ref, bg_ref, h1_ref, x2_ref, t2_ref):
        g1 = jax.nn.relu(s1_ref[0] + s1_ref[1])
        t1 = t1_ref[...]
        h1 = t1 * g1 + (1.0 - t1) * nr_ref[...]
        h1_ref[...] = h1
        x2_ref[...] = jnp.dot(h1, w2_ref[...], preferred_element_type=jnp.float32)
        t2_ref[...] = jax.nn.sigmoid(
            jnp.dot(h1, kg_ref[...], preferred_element_type=jnp.float32) + bg_ref[...])

    return _tc_call(
        body,
        (jax.ShapeDtypeStruct((E_NODES, D), jnp.float32),
         jax.ShapeDtypeStruct((E_NODES, D), jnp.float32),
         jax.ShapeDtypeStruct((E_NODES, D), jnp.float32)),
        s1_part, nr, t1, w2, kg, bg,
    )


def _tc_node(s2_part, h1, t2):
    def body(s2_ref, h1_ref, t2_ref, node_ref):
        g2 = jax.nn.relu(s2_ref[0] + s2_ref[1])
        t2 = t2_ref[...]
        node_ref[...] = t2 * g2 + (1.0 - t2) * h1_ref[...]

    return _tc_call(
        body,
        jax.ShapeDtypeStruct((E_NODES, D), jnp.float32),
        s2_part, h1, t2,
    )


L_OFF = 0
R_OFF = 5000
NR_OFF = 10000
NL_OFF = 85000


def _tc_loss(g, mask_col, T, K):
    TB = 200
    TBK = TB * K
    nblk = T // TB

    def body(l_ref, r_ref, nr_ref, nl_ref, m_ref, o_ref):
        i = pl.program_id(0)
        l = l_ref[...]
        r = r_ref[...]
        sel = (jax.lax.broadcasted_iota(jnp.int32, (TBK, TB), 0) // K
               == jax.lax.broadcasted_iota(jnp.int32, (TBK, TB), 1)).astype(jnp.float32)
        A = jnp.sum(jnp.abs(l - r), axis=1, keepdims=True)
        dm_rep = jnp.dot(sel, A + GAMMA, preferred_element_type=jnp.float32)
        l_rep = jnp.dot(sel, l, preferred_element_type=jnp.float32)
        r_rep = jnp.dot(sel, r, preferred_element_type=jnp.float32)
        B = jnp.sum(jnp.abs(l_rep - nr_ref[...]), axis=1, keepdims=True)
        B2 = jnp.sum(jnp.abs(nl_ref[...] - r_rep), axis=1, keepdims=True)
        m = m_ref[...]
        terms = (jax.nn.relu(dm_rep - B) + jax.nn.relu(dm_rep - B2)) * m
        part = (jnp.sum(terms) / 2.0).reshape(1, 1)

        @pl.when(i == 0)
        def _():
            o_ref[...] = jnp.zeros_like(o_ref)

        o_ref[...] += part

    out = pl.pallas_call(
        body,
        grid=(nblk,),
        in_specs=[
            pl.BlockSpec((TB, D), lambda i: (i + L_OFF // TB, 0)),
            pl.BlockSpec((TB, D), lambda i: (i + R_OFF // TB, 0)),
            pl.BlockSpec((TBK, D), lambda i: (i + NR_OFF // TBK, 0)),
            pl.BlockSpec((TBK, D), lambda i: (i + NL_OFF // TBK, 0)),
            pl.BlockSpec((TBK, 1), lambda i: (i, 0)),
        ],
        out_specs=pl.BlockSpec((1, 1), lambda i: (0, 0)),
        out_shape=jax.ShapeDtypeStruct((1, 1), jnp.float32),
    )(g, g, g, g, mask_col)
    return out[0, 0]


def kernel(left_idx, right_idx, neg_right, neg_left, head_rows, head_cols, head_vals, tail_rows, tail_cols, tail_vals, er_rows, er_cols, er_vals, adj_rows, adj_cols, adj_vals, mask, word_emb, kernel_gate, bias_gate, W1, W2, Dense, Bias):
    f32 = jnp.float32
    i32 = jnp.int32
    zeros_nodes = jnp.zeros((E_NODES, D), f32)
    zeros_rel = jnp.zeros((N_REL, D), f32)

    hr, hc, hv = _pad2d(head_rows.astype(i32), 0, SLAB_ST), _pad2d(head_cols.astype(i32), 0, SLAB_ST), _pad2d(head_vals, 0.0, SLAB_ST)
    tr, tc, tv = _pad2d(tail_rows.astype(i32), 0, SLAB_ST), _pad2d(tail_cols.astype(i32), 0, SLAB_ST), _pad2d(tail_vals, 0.0, SLAB_ST)
    err, erc, erv = _pad2d(er_rows.astype(i32), 0, SLAB_ST), _pad2d(er_cols.astype(i32), 0, SLAB_ST), _pad2d(er_vals, 0.0, SLAB_ST)
    ar, ac, av = _pad2d(adj_rows.astype(i32), 0), _pad2d(adj_cols.astype(i32), 0), _pad2d(adj_vals, 0.0)

    we, p = _tc_norm_p(word_emb, Dense[:D], Bias.reshape(1, D))

    lr_part = _headtail_sc(hr, hc, hv, tr, tc, tv, we, zeros_rel)
    z = _tc_z(lr_part, Dense[D:2 * D], Dense[2 * D:])

    nb_part = _spmm_sc(err, erc, erv, z, E_NODES, zeros_nodes, stage_table=True)
    nr, x1, t1 = _tc_nr(nb_part, we, p, W1, kernel_gate, bias_gate.reshape(1, D))

    s1_part = _spmm_sc(ar, ac, av, x1, E_NODES, zeros_nodes, stage_table=False)
    h1, x2, t2 = _tc_h1(s1_part, nr, t1, W2, kernel_gate, bias_gate.reshape(1, D))

    s2_part = _spmm_sc(ar, ac, av, x2, E_NODES, zeros_nodes, stage_table=False)
    node = _tc_node(s2_part, h1, t2)

    t_pairs, k_neg = neg_right.shape
    all_idx = jnp.zeros((NL_OFF + t_pairs * k_neg,), i32)
    all_idx = all_idx.at[L_OFF:L_OFF + t_pairs].set(left_idx.astype(i32))
    all_idx = all_idx.at[R_OFF:R_OFF + t_pairs].set(right_idx.astype(i32))
    all_idx = all_idx.at[NR_OFF:NR_OFF + t_pairs * k_neg].set(neg_right.astype(i32).reshape(-1))
    all_idx = all_idx.at[NL_OFF:].set(neg_left.astype(i32).reshape(-1))
    idx2d = _pad2d(all_idx, 0)
    g = _gather_sc(idx2d, node)
    return _tc_loss(g, mask.reshape(t_pairs * k_neg, 1), t_pairs, k_neg)
